# baseline (speedup 1.0000x reference)
import functools

import jax
import jax.numpy as jnp
from jax import lax
from jax.experimental import pallas as pl
from jax.experimental.pallas import tpu as pltpu
from jax.experimental.pallas import tpu_sc as plsc

N = 10000
D = 128
H = 128
NMP = 3
NC_OUT = 16
DA = 64
E = 320000

NTILES = 16
EPT = 20480
NBLK = 160
EPAD = EPT * NTILES
NPAD = N + 112
RPT = NPAD // NTILES
CHUNKS = tuple((o, min(128, RPT - o)) for o in range(0, RPT, 128))
IBLK = 8
NCHUNK = NBLK // IBLK
NPAIR = NCHUNK // 2
IROWS = EPAD // 128


def _sc_segment_sum(feat1, feat2, src2d, dst2d, zrows, zones):
    mesh = plsc.VectorSubcoreMesh(core_axis_name="c", subcore_axis_name="s")

    @functools.partial(
        pl.kernel,
        mesh=mesh,
        compiler_params=pltpu.CompilerParams(use_tc_tiling_on_sc=False),
        out_type=[
            jax.ShapeDtypeStruct((NPAD, D), jnp.float32),
            jax.ShapeDtypeStruct((NPAD, D), jnp.float32),
            jax.ShapeDtypeStruct((NPAD, 16), jnp.float32),
            jax.ShapeDtypeStruct((NPAD, 16), jnp.float32),
        ],
        scratch_types=[
            pltpu.VMEM((IBLK, 128), jnp.int32),
            pltpu.VMEM((IBLK, 128), jnp.int32),
            pltpu.VMEM((IBLK, 128), jnp.int32),
            pltpu.VMEM((IBLK, 128), jnp.int32),
            pltpu.VMEM((128, D), jnp.float32),
            pltpu.VMEM((128, D), jnp.float32),
            pltpu.VMEM((128, 16), jnp.float32),
            pltpu.VMEM_SHARED((NPAD, D), jnp.float32),
            pltpu.VMEM_SHARED((NPAD, 16), jnp.float32),
            pltpu.SemaphoreType.DMA,
            pltpu.SemaphoreType.DMA,
            pltpu.SemaphoreType.DMA,
            pltpu.SemaphoreType.DMA,
            pltpu.SemaphoreType.DMA,
            pltpu.SemaphoreType.DMA,
            pltpu.SemaphoreType.DMA,
        ],
    )
    def run(feat1_hbm, feat2_hbm, src_hbm, dst_hbm, zrows_hbm, zones_hbm,
            agg1_hbm, agg2_hbm, deg1_hbm, deg2_hbm,
            src_a, dst_a, src_b, dst_b, rows_v0, rows_v1, ones_v,
            agg_sp, deg_sp, sem0a, sem0b, sem1a, sem1b, sem_ia, sem_ib,
            sem_d):
        c = lax.axis_index("c")
        s = lax.axis_index("s")
        rbase = s * RPT

        pltpu.sync_copy(zrows_hbm, rows_v1)
        pltpu.sync_copy(zones_hbm.at[pl.ds(128, 128)], ones_v)
        for off, nr in CHUNKS:
            pltpu.sync_copy(rows_v1.at[pl.ds(0, nr)],
                            agg_sp.at[pl.ds(rbase + off, nr)])
            pltpu.sync_copy(ones_v.at[pl.ds(0, nr)],
                            deg_sp.at[pl.ds(rbase + off, nr)])
        pltpu.sync_copy(zones_hbm.at[pl.ds(0, 128)], ones_v)

        plsc.subcore_barrier()

        rows = (rows_v0, rows_v1)
        sems = ((sem0a, sem0b), (sem1a, sem1b))
        HALF = 64

        def issue_rows(feat_hbm, sv, j, k):
            pltpu.async_copy(feat_hbm.at[sv.at[j]], rows[k], sems[k][0])

        def wait_rows(feat_hbm, k):
            pltpu.make_async_copy(feat_hbm.at[pl.ds(0, 128)],
                                  rows[k], sems[k][0]).wait()

        def wait_idx(sbuf, dbuf, sem):
            pltpu.make_async_copy(src_hbm.at[pl.ds(0, IBLK)], sbuf, sem).wait()
            pltpu.make_async_copy(dst_hbm.at[pl.ds(0, IBLK)], dbuf, sem).wait()

        def issue_idx(g, sbuf, dbuf, sem):
            base = jnp.minimum(s * NBLK + g * IBLK, IROWS - IBLK)
            pltpu.async_copy(src_hbm.at[pl.ds(base, IBLK)], sbuf, sem)
            pltpu.async_copy(dst_hbm.at[pl.ds(base, IBLK)], dbuf, sem)

        def wait_deg():
            pltpu.make_async_copy(zones_hbm.at[pl.ds(0, 128)], ones_v,
                                  sem_d).wait()

        def make_pipeline(feat_hbm, deg_first_half):
            def half(sv, dv, other_sv, other_sem, do_deg):
                for j in range(IBLK):
                    if j + 1 < IBLK:
                        issue_rows(feat_hbm, sv, j + 1, (j + 1) % 2)
                    else:
                        wait_idx(other_sv[0], other_sv[1], other_sem)
                        issue_rows(feat_hbm, other_sv[0], 0, 0)
                    wait_rows(feat_hbm, j % 2)
                    pltpu.sync_copy(rows[j % 2], agg_sp.at[dv.at[j]],
                                    add=True)

                    @pl.when(do_deg)
                    def _():
                        if j > 0:
                            wait_deg()
                        pltpu.async_copy(ones_v, deg_sp.at[dv.at[j]], sem_d,
                                         add=True)

                @pl.when(do_deg)
                def _():
                    wait_deg()

            def pair(p, carry):
                if deg_first_half:
                    do_deg = p < NPAIR // 2
                else:
                    do_deg = p >= NPAIR // 2
                half(src_a, dst_a, (src_b, dst_b), sem_ib, do_deg)
                issue_idx(2 * p + 2, src_a, dst_a, sem_ia)
                half(src_b, dst_b, (src_a, dst_a), sem_ia, do_deg)
                issue_idx(2 * p + 3, src_b, dst_b, sem_ib)
                return carry

            pltpu.sync_copy(src_hbm.at[pl.ds(s * NBLK, IBLK)], src_a)
            pltpu.sync_copy(dst_hbm.at[pl.ds(s * NBLK, IBLK)], dst_a)
            pltpu.async_copy(
                src_hbm.at[pl.ds(s * NBLK + IBLK, IBLK)], src_b, sem_ib)
            pltpu.async_copy(
                dst_hbm.at[pl.ds(s * NBLK + IBLK, IBLK)], dst_b, sem_ib)
            issue_rows(feat_hbm, src_a, 0, 0)

            lax.fori_loop(0, NPAIR, pair, 0)

            wait_rows(feat_hbm, 0)
            wait_idx(src_b, dst_b, sem_ib)

        @pl.when(c == 0)
        def _():
            make_pipeline(feat1_hbm, True)

        @pl.when(c == 1)
        def _():
            make_pipeline(feat2_hbm, False)

        plsc.subcore_barrier()

        def copy_out(agg_hbm, deg_hbm):
            for off, nr in CHUNKS:
                pltpu.sync_copy(agg_sp.at[pl.ds(rbase + off, nr)],
                                rows_v1.at[pl.ds(0, nr)])
                pltpu.sync_copy(rows_v1.at[pl.ds(0, nr)],
                                agg_hbm.at[pl.ds(rbase + off, nr)])
                pltpu.sync_copy(deg_sp.at[pl.ds(rbase + off, nr)],
                                ones_v.at[pl.ds(0, nr)])
                pltpu.sync_copy(ones_v.at[pl.ds(0, nr)],
                                deg_hbm.at[pl.ds(rbase + off, nr)])

        @pl.when(c == 0)
        def _():
            copy_out(agg1_hbm, deg1_hbm)

        @pl.when(c == 1)
        def _():
            copy_out(agg2_hbm, deg2_hbm)

    return run(feat1, feat2, src2d, dst2d, zrows, zones)


BLK = 2000
GRID = N // BLK


def _tc_attn_body(a1_ref, a2_ref, d1_ref, d2_ref, wg_ref, wa_ref, msk_ref,
                  sacc_ref, racc_ref):
    i = pl.program_id(0)

    @pl.when(i == 0)
    def _():
        sacc_ref[...] = jnp.zeros_like(sacc_ref)
        racc_ref[...] = jnp.zeros_like(racc_ref)

    deg = d1_ref[:, :1] + d2_ref[:, :1]
    dinv = 1.0 / jnp.maximum(deg, 1.0)
    x1 = a1_ref[...] * dinv
    x2 = a2_ref[...] * dinv
    wa = wa_ref[...]
    msk = msk_ref[0]
    for m in range(NMP):
        w = wg_ref[m]
        h1 = jnp.maximum(jnp.dot(x1, w, preferred_element_type=jnp.float32), 0.0)
        t1 = jnp.tanh(jnp.dot(h1, wa, preferred_element_type=jnp.float32))
        sacc_ref[m:m + 1, :] = sacc_ref[m:m + 1, :] + jnp.sum(t1, axis=0, keepdims=True)
        racc_ref[m:m + 1, :] = racc_ref[m:m + 1, :] + jnp.dot(
            msk, h1, preferred_element_type=jnp.float32)
        h2 = jnp.maximum(jnp.dot(x2, w, preferred_element_type=jnp.float32), 0.0)
        t2 = jnp.tanh(jnp.dot(h2, wa, preferred_element_type=jnp.float32))
        sacc_ref[NMP + m:NMP + m + 1, :] = (
            sacc_ref[NMP + m:NMP + m + 1, :] + jnp.sum(t2, axis=0, keepdims=True))
    racc_ref[NMP:NMP + 1, :] = racc_ref[NMP:NMP + 1, :] + jnp.full(
        (1, H), jnp.sum(msk))


def _softmax3(s0, s1, s2):
    mx = jnp.maximum(s0, jnp.maximum(s1, s2))
    e0 = jnp.exp(s0 - mx)
    e1 = jnp.exp(s1 - mx)
    e2 = jnp.exp(s2 - mx)
    tot = e0 + e1 + e2
    return e0 / tot, e1 / tot, e2 / tot


def _tc_final_body(a1_ref, a2_ref, d1_ref, d2_ref, sacc_ref, avec_ref,
                   racc_ref, wg_ref, wfc_ref, bfc_ref, wdt_ref,
                   preds_ref, sc1_ref, sc2_ref, beta_ref):
    sv = jnp.sum(sacc_ref[...] * avec_ref[...], axis=1, keepdims=True) / N
    b10, b11, b12 = _softmax3(sv[0, 0], sv[1, 0], sv[2, 0])
    b20, b21, b22 = _softmax3(sv[3, 0], sv[4, 0], sv[5, 0])

    c_num = (b10 * racc_ref[0:1, :] + b11 * racc_ref[1:2, :]
             + b12 * racc_ref[2:3, :])
    c_row = jax.nn.sigmoid(c_num / racc_ref[NMP:NMP + 1, :])
    u_row = jnp.dot(c_row, wdt_ref[...], preferred_element_type=jnp.float32)

    deg = d1_ref[:, :1] + d2_ref[:, :1]
    dinv = 1.0 / jnp.maximum(deg, 1.0)
    x1 = a1_ref[...] * dinv
    x2 = a2_ref[...] * dinv
    beta1 = (b10, b11, b12)
    beta2 = (b20, b21, b22)
    h1c = jnp.zeros((BLK, H), jnp.float32)
    h2c = jnp.zeros((BLK, H), jnp.float32)
    for m in range(NMP):
        w = wg_ref[m]
        h1c = h1c + beta1[m] * jnp.maximum(
            jnp.dot(x1, w, preferred_element_type=jnp.float32), 0.0)
        h2c = h2c + beta2[m] * jnp.maximum(
            jnp.dot(x2, w, preferred_element_type=jnp.float32), 0.0)
    preds_ref[...] = jnp.dot(h1c, wfc_ref[...],
                             preferred_element_type=jnp.float32) + bfc_ref[...]
    s1 = jnp.sum(h1c * u_row, axis=1, keepdims=True)
    s2 = jnp.sum(h2c * u_row, axis=1, keepdims=True)
    sc1_ref[...] = jnp.broadcast_to(s1, (BLK, 8))
    sc2_ref[...] = jnp.broadcast_to(s2, (BLK, 8))

    lanes = lax.broadcasted_iota(jnp.int32, (1, 128), 1)
    beta_ref[0:1, :] = jnp.where(
        lanes == 0, b10, jnp.where(lanes == 1, b11,
                                   jnp.where(lanes == 2, b12, 0.0)))


def kernel(feat1, feat2, msk, samp_bias1, samp_bias2, edge_index, W_gnn, Wa,
           a_vec, W_fc, b_fc, W_disc):
    f32 = jnp.float32
    src = edge_index[0]
    dst = edge_index[1]
    npad = EPAD - E
    pad_idx = jnp.arange(npad, dtype=jnp.int32)
    src_p = jnp.concatenate([src, pad_idx % N])
    dst_p = jnp.concatenate([dst, N + (pad_idx % 112)])
    src2d = src_p.reshape(EPAD // 128, 128)
    dst2d = dst_p.reshape(EPAD // 128, 128)
    zrows = jnp.zeros((128, D), f32)
    zones = jnp.concatenate([jnp.ones((128, 16), f32),
                             jnp.zeros((128, 16), f32)], axis=0)

    agg1, agg2, deg1, deg2 = _sc_segment_sum(feat1, feat2, src2d, dst2d,
                                             zrows, zones)

    blk_spec = pl.BlockSpec((BLK, D), lambda i: (i, 0))
    deg_spec = pl.BlockSpec((BLK, 16), lambda i: (i, 0))
    sacc, racc = pl.pallas_call(
        _tc_attn_body,
        grid=(GRID,),
        in_specs=[
            blk_spec,
            blk_spec,
            deg_spec,
            deg_spec,
            pl.BlockSpec((NMP, D, H), lambda i: (0, 0, 0)),
            pl.BlockSpec((H, DA), lambda i: (0, 0)),
            pl.BlockSpec((1, 1, BLK), lambda i: (i, 0, 0)),
        ],
        out_specs=[
            pl.BlockSpec((8, DA), lambda i: (0, 0)),
            pl.BlockSpec((8, H), lambda i: (0, 0)),
        ],
        out_shape=[
            jax.ShapeDtypeStruct((8, DA), f32),
            jax.ShapeDtypeStruct((8, H), f32),
        ],
    )(agg1, agg2, deg1, deg2, W_gnn, Wa, msk.reshape(GRID, 1, BLK))

    preds, sc1, sc2, beta = pl.pallas_call(
        _tc_final_body,
        grid=(GRID,),
        in_specs=[
            blk_spec,
            blk_spec,
            deg_spec,
            deg_spec,
            pl.BlockSpec((8, DA), lambda i: (0, 0)),
            pl.BlockSpec((1, DA), lambda i: (0, 0)),
            pl.BlockSpec((8, H), lambda i: (0, 0)),
            pl.BlockSpec((NMP, D, H), lambda i: (0, 0, 0)),
            pl.BlockSpec((H, NC_OUT), lambda i: (0, 0)),
            pl.BlockSpec((1, NC_OUT), lambda i: (0, 0)),
            pl.BlockSpec((H, H), lambda i: (0, 0)),
        ],
        out_specs=[
            pl.BlockSpec((BLK, NC_OUT), lambda i: (i, 0)),
            pl.BlockSpec((BLK, 8), lambda i: (i, 0)),
            pl.BlockSpec((BLK, 8), lambda i: (i, 0)),
            pl.BlockSpec((1, 128), lambda i: (0, 0)),
        ],
        out_shape=[
            jax.ShapeDtypeStruct((N, NC_OUT), f32),
            jax.ShapeDtypeStruct((N, 8), f32),
            jax.ShapeDtypeStruct((N, 8), f32),
            jax.ShapeDtypeStruct((1, 128), f32),
        ],
    )(agg1, agg2, deg1, deg2, sacc, a_vec.reshape(1, DA), racc, W_gnn,
      W_fc, b_fc.reshape(1, NC_OUT), W_disc.T)

    weights = beta[0, :NMP]
    reg = jnp.concatenate([sc1[:, 0] + samp_bias1[0],
                           sc2[:, 0] + samp_bias2[0]])[None, :]
    return (preds, weights, reg)


# speedup vs baseline: 1.0030x; 1.0030x over previous
import functools

import jax
import jax.numpy as jnp
from jax import lax
from jax.experimental import pallas as pl
from jax.experimental.pallas import tpu as pltpu
from jax.experimental.pallas import tpu_sc as plsc

N = 10000
D = 128
H = 128
NMP = 3
NC_OUT = 16
DA = 64
E = 320000

NTILES = 16
EPT = 20480
NBLK = 160
EPAD = EPT * NTILES
NPAD = N + 112
RPT = NPAD // NTILES
CHUNKS = tuple((o, min(128, RPT - o)) for o in range(0, RPT, 128))
IBLK = 8
NCHUNK = NBLK // IBLK
NPAIR = NCHUNK // 2
IROWS = EPAD // 128


def _sc_segment_sum(feat1, feat2, src2d, dst2d, zrows, zones):
    mesh = plsc.VectorSubcoreMesh(core_axis_name="c", subcore_axis_name="s")

    @functools.partial(
        pl.kernel,
        mesh=mesh,
        compiler_params=pltpu.CompilerParams(use_tc_tiling_on_sc=False),
        out_type=[
            jax.ShapeDtypeStruct((NPAD, D), jnp.float32),
            jax.ShapeDtypeStruct((NPAD, D), jnp.float32),
            jax.ShapeDtypeStruct((NPAD, 16), jnp.float32),
            jax.ShapeDtypeStruct((NPAD, 16), jnp.float32),
        ],
        scratch_types=[
            pltpu.VMEM((IBLK, 128), jnp.int32),
            pltpu.VMEM((IBLK, 128), jnp.int32),
            pltpu.VMEM((IBLK, 128), jnp.int32),
            pltpu.VMEM((IBLK, 128), jnp.int32),
            pltpu.VMEM((128, D), jnp.float32),
            pltpu.VMEM((128, D), jnp.float32),
            pltpu.VMEM((128, 16), jnp.float32),
            pltpu.VMEM_SHARED((NPAD, D), jnp.float32),
            pltpu.VMEM_SHARED((NPAD, 16), jnp.float32),
            pltpu.SemaphoreType.DMA,
            pltpu.SemaphoreType.DMA,
            pltpu.SemaphoreType.DMA,
            pltpu.SemaphoreType.DMA,
            pltpu.SemaphoreType.DMA,
            pltpu.SemaphoreType.DMA,
            pltpu.SemaphoreType.DMA,
        ],
    )
    def run(feat1_hbm, feat2_hbm, src_hbm, dst_hbm, zrows_hbm, zones_hbm,
            agg1_hbm, agg2_hbm, deg1_hbm, deg2_hbm,
            src_a, dst_a, src_b, dst_b, rows_v0, rows_v1, ones_v,
            agg_sp, deg_sp, sem0a, sem0b, sem1a, sem1b, sem_ia, sem_ib,
            sem_d):
        c = lax.axis_index("c")
        s = lax.axis_index("s")
        rbase = s * RPT

        pltpu.sync_copy(zrows_hbm, rows_v1)
        pltpu.sync_copy(zones_hbm.at[pl.ds(128, 128)], ones_v)
        for off, nr in CHUNKS:
            pltpu.sync_copy(rows_v1.at[pl.ds(0, nr)],
                            agg_sp.at[pl.ds(rbase + off, nr)])
            pltpu.sync_copy(ones_v.at[pl.ds(0, nr)],
                            deg_sp.at[pl.ds(rbase + off, nr)])
        pltpu.sync_copy(zones_hbm.at[pl.ds(0, 128)], ones_v)

        plsc.subcore_barrier()

        rows = (rows_v0, rows_v1)
        sems = ((sem0a, sem0b), (sem1a, sem1b))

        def issue_rows(feat_hbm, sv, j, k):
            pltpu.async_copy(feat_hbm.at[sv.at[j]], rows[k], sems[k][0])

        def wait_rows(feat_hbm, k):
            pltpu.make_async_copy(feat_hbm.at[pl.ds(0, 128)],
                                  rows[k], sems[k][0]).wait()

        def wait_idx(sbuf, dbuf, sem):
            pltpu.make_async_copy(src_hbm.at[pl.ds(0, IBLK)], sbuf, sem).wait()
            pltpu.make_async_copy(dst_hbm.at[pl.ds(0, IBLK)], dbuf, sem).wait()

        def issue_idx(g, sbuf, dbuf, sem):
            base = jnp.minimum(s * NBLK + g * IBLK, IROWS - IBLK)
            pltpu.async_copy(src_hbm.at[pl.ds(base, IBLK)], sbuf, sem)
            pltpu.async_copy(dst_hbm.at[pl.ds(base, IBLK)], dbuf, sem)

        def wait_deg():
            pltpu.make_async_copy(zones_hbm.at[pl.ds(0, 128)], ones_v,
                                  sem_d).wait()

        def make_pipeline(feat_hbm, deg_first_half):
            def half(sv, dv, other_sv, other_sem, do_deg):
                for j in range(IBLK):
                    if j + 1 < IBLK:
                        issue_rows(feat_hbm, sv, j + 1, (j + 1) % 2)
                    else:
                        wait_idx(other_sv[0], other_sv[1], other_sem)
                        issue_rows(feat_hbm, other_sv[0], 0, 0)
                    wait_rows(feat_hbm, j % 2)
                    pltpu.sync_copy(rows[j % 2], agg_sp.at[dv.at[j]],
                                    add=True)

                    @pl.when(do_deg)
                    def _():
                        if j > 0:
                            wait_deg()
                        pltpu.async_copy(ones_v, deg_sp.at[dv.at[j]], sem_d,
                                         add=True)

                @pl.when(do_deg)
                def _():
                    wait_deg()

            def pair(p, carry):
                if deg_first_half:
                    do_deg = p < NPAIR // 2
                else:
                    do_deg = p >= NPAIR // 2
                half(src_a, dst_a, (src_b, dst_b), sem_ib, do_deg)
                issue_idx(2 * p + 2, src_a, dst_a, sem_ia)
                half(src_b, dst_b, (src_a, dst_a), sem_ia, do_deg)
                issue_idx(2 * p + 3, src_b, dst_b, sem_ib)
                return carry

            pltpu.sync_copy(src_hbm.at[pl.ds(s * NBLK, IBLK)], src_a)
            pltpu.sync_copy(dst_hbm.at[pl.ds(s * NBLK, IBLK)], dst_a)
            pltpu.async_copy(
                src_hbm.at[pl.ds(s * NBLK + IBLK, IBLK)], src_b, sem_ib)
            pltpu.async_copy(
                dst_hbm.at[pl.ds(s * NBLK + IBLK, IBLK)], dst_b, sem_ib)
            issue_rows(feat_hbm, src_a, 0, 0)

            lax.fori_loop(0, NPAIR, pair, 0)

            wait_rows(feat_hbm, 0)
            wait_idx(src_b, dst_b, sem_ib)

        @pl.when(c == 0)
        def _():
            make_pipeline(feat1_hbm, True)

        @pl.when(c == 1)
        def _():
            make_pipeline(feat2_hbm, False)

        plsc.subcore_barrier()

        def copy_out(agg_hbm, deg_hbm):
            for off, nr in CHUNKS:
                pltpu.sync_copy(agg_sp.at[pl.ds(rbase + off, nr)],
                                rows_v1.at[pl.ds(0, nr)])
                pltpu.sync_copy(rows_v1.at[pl.ds(0, nr)],
                                agg_hbm.at[pl.ds(rbase + off, nr)])
                pltpu.sync_copy(deg_sp.at[pl.ds(rbase + off, nr)],
                                ones_v.at[pl.ds(0, nr)])
                pltpu.sync_copy(ones_v.at[pl.ds(0, nr)],
                                deg_hbm.at[pl.ds(rbase + off, nr)])

        @pl.when(c == 0)
        def _():
            copy_out(agg1_hbm, deg1_hbm)

        @pl.when(c == 1)
        def _():
            copy_out(agg2_hbm, deg2_hbm)

    return run(feat1, feat2, src2d, dst2d, zrows, zones)


BLK = 2000
GRID = N // BLK


def _tc_attn_body(a1_ref, a2_ref, d1_ref, d2_ref, wg_ref, wa_ref, msk_ref,
                  sacc_ref, racc_ref):
    i = pl.program_id(0)

    @pl.when(i == 0)
    def _():
        sacc_ref[...] = jnp.zeros_like(sacc_ref)
        racc_ref[...] = jnp.zeros_like(racc_ref)

    deg = d1_ref[:, :1] + d2_ref[:, :1]
    dinv = 1.0 / jnp.maximum(deg, 1.0)
    x1 = a1_ref[...] * dinv
    x2 = a2_ref[...] * dinv
    wa = wa_ref[...]
    msk = msk_ref[0]
    for m in range(NMP):
        w = wg_ref[m]
        h1 = jnp.maximum(jnp.dot(x1, w, preferred_element_type=jnp.float32), 0.0)
        t1 = jnp.tanh(jnp.dot(h1, wa, preferred_element_type=jnp.float32))
        sacc_ref[m:m + 1, :] = sacc_ref[m:m + 1, :] + jnp.sum(t1, axis=0, keepdims=True)
        racc_ref[m:m + 1, :] = racc_ref[m:m + 1, :] + jnp.dot(
            msk, h1, preferred_element_type=jnp.float32)
        h2 = jnp.maximum(jnp.dot(x2, w, preferred_element_type=jnp.float32), 0.0)
        t2 = jnp.tanh(jnp.dot(h2, wa, preferred_element_type=jnp.float32))
        sacc_ref[NMP + m:NMP + m + 1, :] = (
            sacc_ref[NMP + m:NMP + m + 1, :] + jnp.sum(t2, axis=0, keepdims=True))
    racc_ref[NMP:NMP + 1, :] = racc_ref[NMP:NMP + 1, :] + jnp.full(
        (1, H), jnp.sum(msk))


def _softmax3(s0, s1, s2):
    mx = jnp.maximum(s0, jnp.maximum(s1, s2))
    e0 = jnp.exp(s0 - mx)
    e1 = jnp.exp(s1 - mx)
    e2 = jnp.exp(s2 - mx)
    tot = e0 + e1 + e2
    return e0 / tot, e1 / tot, e2 / tot


def _tc_final_body(a1_ref, a2_ref, d1_ref, d2_ref, sacc_ref, avec_ref,
                   racc_ref, wg_ref, wfc_ref, bfc_ref, wdt_ref,
                   preds_ref, sc1_ref, sc2_ref, beta_ref):
    sv = jnp.sum(sacc_ref[...] * avec_ref[...], axis=1, keepdims=True) / N
    b10, b11, b12 = _softmax3(sv[0, 0], sv[1, 0], sv[2, 0])
    b20, b21, b22 = _softmax3(sv[3, 0], sv[4, 0], sv[5, 0])

    c_num = (b10 * racc_ref[0:1, :] + b11 * racc_ref[1:2, :]
             + b12 * racc_ref[2:3, :])
    c_row = jax.nn.sigmoid(c_num / racc_ref[NMP:NMP + 1, :])
    u_row = jnp.dot(c_row, wdt_ref[...], preferred_element_type=jnp.float32)

    deg = d1_ref[:, :1] + d2_ref[:, :1]
    dinv = 1.0 / jnp.maximum(deg, 1.0)
    x1 = a1_ref[...] * dinv
    x2 = a2_ref[...] * dinv
    beta1 = (b10, b11, b12)
    beta2 = (b20, b21, b22)
    h1c = jnp.zeros((BLK, H), jnp.float32)
    h2c = jnp.zeros((BLK, H), jnp.float32)
    for m in range(NMP):
        w = wg_ref[m]
        h1c = h1c + beta1[m] * jnp.maximum(
            jnp.dot(x1, w, preferred_element_type=jnp.float32), 0.0)
        h2c = h2c + beta2[m] * jnp.maximum(
            jnp.dot(x2, w, preferred_element_type=jnp.float32), 0.0)
    preds_ref[...] = jnp.dot(h1c, wfc_ref[...],
                             preferred_element_type=jnp.float32) + bfc_ref[...]
    s1 = jnp.sum(h1c * u_row, axis=1, keepdims=True)
    s2 = jnp.sum(h2c * u_row, axis=1, keepdims=True)
    sc1_ref[...] = jnp.broadcast_to(s1, (BLK, 8))
    sc2_ref[...] = jnp.broadcast_to(s2, (BLK, 8))

    lanes = lax.broadcasted_iota(jnp.int32, (1, 128), 1)
    beta_ref[0:1, :] = jnp.where(
        lanes == 0, b10, jnp.where(lanes == 1, b11,
                                   jnp.where(lanes == 2, b12, 0.0)))


def kernel(feat1, feat2, msk, samp_bias1, samp_bias2, edge_index, W_gnn, Wa,
           a_vec, W_fc, b_fc, W_disc):
    f32 = jnp.float32
    src = edge_index[0]
    dst = edge_index[1]
    npad = EPAD - E
    pad_idx = jnp.arange(npad, dtype=jnp.int32)
    src_p = jnp.concatenate([src, pad_idx % N])
    dst_p = jnp.concatenate([dst, N + (pad_idx % 112)])
    src2d = src_p.reshape(EPAD // 128, 128)
    dst2d = dst_p.reshape(EPAD // 128, 128)
    zrows = jnp.zeros((128, D), f32)
    zones = jnp.concatenate([jnp.ones((128, 16), f32),
                             jnp.zeros((128, 16), f32)], axis=0)

    agg1, agg2, deg1, deg2 = _sc_segment_sum(feat1, feat2, src2d, dst2d,
                                             zrows, zones)

    blk_spec = pl.BlockSpec((BLK, D), lambda i: (i, 0))
    deg_spec = pl.BlockSpec((BLK, 16), lambda i: (i, 0))
    sacc, racc = pl.pallas_call(
        _tc_attn_body,
        grid=(GRID,),
        in_specs=[
            blk_spec,
            blk_spec,
            deg_spec,
            deg_spec,
            pl.BlockSpec((NMP, D, H), lambda i: (0, 0, 0)),
            pl.BlockSpec((H, DA), lambda i: (0, 0)),
            pl.BlockSpec((1, 1, BLK), lambda i: (i, 0, 0)),
        ],
        out_specs=[
            pl.BlockSpec((8, DA), lambda i: (0, 0)),
            pl.BlockSpec((8, H), lambda i: (0, 0)),
        ],
        out_shape=[
            jax.ShapeDtypeStruct((8, DA), f32),
            jax.ShapeDtypeStruct((8, H), f32),
        ],
    )(agg1, agg2, deg1, deg2, W_gnn, Wa, msk.reshape(GRID, 1, BLK))

    preds, sc1, sc2, beta = pl.pallas_call(
        _tc_final_body,
        grid=(GRID,),
        in_specs=[
            blk_spec,
            blk_spec,
            deg_spec,
            deg_spec,
            pl.BlockSpec((8, DA), lambda i: (0, 0)),
            pl.BlockSpec((1, DA), lambda i: (0, 0)),
            pl.BlockSpec((8, H), lambda i: (0, 0)),
            pl.BlockSpec((NMP, D, H), lambda i: (0, 0, 0)),
            pl.BlockSpec((H, NC_OUT), lambda i: (0, 0)),
            pl.BlockSpec((1, NC_OUT), lambda i: (0, 0)),
            pl.BlockSpec((H, H), lambda i: (0, 0)),
        ],
        out_specs=[
            pl.BlockSpec((BLK, NC_OUT), lambda i: (i, 0)),
            pl.BlockSpec((BLK, 8), lambda i: (i, 0)),
            pl.BlockSpec((BLK, 8), lambda i: (i, 0)),
            pl.BlockSpec((1, 128), lambda i: (0, 0)),
        ],
        out_shape=[
            jax.ShapeDtypeStruct((N, NC_OUT), f32),
            jax.ShapeDtypeStruct((N, 8), f32),
            jax.ShapeDtypeStruct((N, 8), f32),
            jax.ShapeDtypeStruct((1, 128), f32),
        ],
    )(agg1, agg2, deg1, deg2, sacc, a_vec.reshape(1, DA), racc, W_gnn,
      W_fc, b_fc.reshape(1, NC_OUT), W_disc.T)

    weights = beta[0, :NMP]
    reg = jnp.concatenate([sc1[:, 0] + samp_bias1[0],
                           sc2[:, 0] + samp_bias2[0]])[None, :]
    return (preds, weights, reg)
